# baseline (speedup 1.0000x reference)
import functools

import jax
import jax.numpy as jnp
from jax import lax
from jax.experimental import pallas as pl
from jax.experimental.pallas import tpu as pltpu
from jax.experimental.pallas import tpu_sc as plsc

N = 10000
E = 320000
D_IN = 128
D_HID = 128
D_OUT = 40
POWER1 = 8

NW = 32
NS = 16
CPW = D_HID // NW
CHUNK = 16000
NCHUNK = E // CHUNK
STEPS = CHUNK // 16
PKC = 2560
NPKC = E // PKC
NPAD = 10112


def _mm_in_body(x_ref, w_ref, b_ref, o_ref):
    o_ref[...] = (
        jnp.dot(x_ref[...], w_ref[...], preferred_element_type=jnp.float32)
        + b_ref[...]
    )


def _lin_in(x, W, b):
    return pl.pallas_call(
        _mm_in_body,
        grid=(10,),
        in_specs=[
            pl.BlockSpec((N // 10, D_IN), lambda i: (i, 0)),
            pl.BlockSpec((D_IN, D_HID), lambda i: (0, 0)),
            pl.BlockSpec((1, D_HID), lambda i: (0, 0)),
        ],
        out_specs=pl.BlockSpec((N // 10, D_HID), lambda i: (i, 0)),
        out_shape=jax.ShapeDtypeStruct((N, D_HID), jnp.float32),
    )(x, W, b.reshape(1, D_HID))


def _prop_body(edge_ref, xxr_ref, out_ref, pk_ref, uxr_ref,
               buf_a, buf_b, dinv, ebuf0, ebuf1, pout, uxbuf, uxbuf2,
               sem0, sem1, sem2, sem3):
    c = lax.axis_index("c")
    s = lax.axis_index("s")
    w = s * 2 + c
    ebufs = [ebuf0, ebuf1]
    sems = [sem0, sem1]
    uxbufs = [uxbuf, uxbuf2]
    uxsems = [sem2, sem3]

    @plsc.parallel_loop(0, N // 16, unroll=8)
    def _(i):
        dinv[pl.ds(i * 16, 16)] = jnp.zeros((16,), jnp.float32)

    ones = jnp.ones((16,), jnp.float32)

    pltpu.async_copy(edge_ref.at[1, pl.ds(0, CHUNK)], ebuf0, sem0)
    pltpu.async_copy(edge_ref.at[1, pl.ds(CHUNK, CHUNK)], ebuf1, sem1)

    def hist_pair(cj, carry):
        for b in range(2):
            ci = cj * 2 + b
            eb = ebufs[b]
            sm = sems[b]
            pltpu.make_async_copy(
                edge_ref.at[1, pl.ds(0, CHUNK)], eb, sm).wait()

            @plsc.parallel_loop(0, STEPS, unroll=16)
            def _(i, eb=eb):
                plsc.addupdate_scatter(dinv, [eb[pl.ds(i * 16, 16)]], ones)

            @pl.when(ci + 2 < NCHUNK)
            def _(eb=eb, sm=sm, ci=ci):
                noff = pl.multiple_of((ci + 2) * CHUNK, 128)
                pltpu.async_copy(
                    edge_ref.at[1, pl.ds(noff, CHUNK)], eb, sm)

        return carry

    lax.fori_loop(0, NCHUNK // 2, hist_pair, 0)

    pltpu.async_copy(edge_ref.at[0, pl.ds(0, CHUNK)], ebuf0, sem0)
    pltpu.async_copy(edge_ref.at[0, pl.ds(CHUNK, CHUNK)], ebuf1, sem1)

    def rmin_pair(cj, rminv):
        for b in range(2):
            ci = cj * 2 + b
            eb = ebufs[b]
            sm = sems[b]
            pltpu.make_async_copy(
                edge_ref.at[0, pl.ds(0, CHUNK)], eb, sm).wait()

            @plsc.parallel_loop(0, STEPS, step=8, carry=rminv)
            def rminv(i, rv, eb=eb):
                vs = [eb[pl.ds((i + k) * 16, 16)] for k in range(8)]
                for st in (4, 2, 1):
                    vs = [jnp.minimum(vs[k], vs[k + st])
                          for k in range(st)]
                return jnp.minimum(rv, vs[0])

            @pl.when(ci + 2 < NCHUNK)
            def _(eb=eb, sm=sm, ci=ci):
                noff = pl.multiple_of((ci + 2) * CHUNK, 128)
                pltpu.async_copy(
                    edge_ref.at[0, pl.ds(noff, CHUNK)], eb, sm)

        return rminv

    rminv = lax.fori_loop(
        0, NCHUNK // 2, rmin_pair,
        jnp.full((16,), jnp.iinfo(jnp.int32).max, jnp.int32),
    )
    rmin = jnp.min(rminv)

    @plsc.parallel_loop(0, N // 16, unroll=5)
    def _(i):
        d = dinv[pl.ds(i * 16, 16)]
        bits = plsc.bitcast(d, jnp.int32)
        y = plsc.bitcast(jnp.int32(0x5F3759DF) - (bits >> 1), jnp.float32)
        nh = d * jnp.float32(-0.5)
        for _ in range(3):
            y = y * (jnp.float32(1.5) + nh * y * y)
        dinv[pl.ds(i * 16, 16)] = jnp.where(d > 0.0, y, jnp.float32(0.0))

    cps = -(-NPKC // NS)
    npk = jnp.minimum(cps, NPKC - s * cps)

    def pack_chunk(k, carry):
        off = pl.multiple_of((s * cps + k) * PKC, 128)
        pltpu.sync_copy(edge_ref.at[0, pl.ds(off, PKC)],
                        ebuf0.at[pl.ds(0, PKC)])
        pltpu.sync_copy(edge_ref.at[1, pl.ds(off, PKC)],
                        ebuf1.at[pl.ds(0, PKC)])

        @plsc.parallel_loop(0, PKC // 16, unroll=4)
        def _(i):
            row16 = ebuf0[pl.ds(i * 16, 16)] - rmin
            col16 = ebuf1[pl.ds(i * 16, 16)]
            pout[pl.ds(i * 16, 16)] = (row16 << 16) | col16

        pltpu.sync_copy(pout, pk_ref.at[c, pl.ds(off, PKC)])
        return carry

    lax.fori_loop(0, npk, pack_chunk, 0)
    plsc.subcore_barrier()

    for b in range(2):
        pltpu.sync_copy(xxr_ref.at[w * CPW + 2 * b], uxbuf)
        pltpu.sync_copy(xxr_ref.at[w * CPW + 2 * b + 1], uxbuf2)

        @plsc.parallel_loop(0, N // 16, unroll=4)
        def _(i, b=b):
            d = dinv[pl.ds(i * 16, 16)]
            v0 = uxbuf[pl.ds(i * 16, 16)] * d
            v1 = uxbuf2[pl.ds(i * 16, 16)] * d
            uxbuf[pl.ds(i * 16, 16)] = v0
            uxbuf2[pl.ds(i * 16, 16)] = v1
            buf_a[pl.ds(b * N + i * 16, 16)] = plsc.bitcast(
                plsc.pack(v0, v1, format=plsc.PackFormat.INTERLEAVED),
                jnp.int32)

        pltpu.sync_copy(uxbuf, uxr_ref.at[w * CPW + 2 * b])
        pltpu.sync_copy(uxbuf2, uxr_ref.at[w * CPW + 2 * b + 1])

    @plsc.parallel_loop(0, CPW * N // 16, unroll=8)
    def _(i):
        buf_b[pl.ds(i * 16, 16)] = jnp.zeros((16,), jnp.float32)

    def half_round(final):
        src = buf_a
        dst = buf_b
        add_ref = xxr_ref if final else uxr_ref
        for j in range(2):
            pltpu.async_copy(add_ref.at[w * CPW + j], uxbufs[j], uxsems[j])
        pltpu.async_copy(pk_ref.at[c, pl.ds(0, CHUNK)], ebuf0, sem0)
        pltpu.async_copy(pk_ref.at[c, pl.ds(CHUNK, CHUNK)], ebuf1, sem1)

        def chunk_pair(cj, carry):
            for b in range(2):
                ci = cj * 2 + b
                eb = ebufs[b]
                sm = sems[b]
                pltpu.make_async_copy(
                    pk_ref.at[c, pl.ds(0, CHUNK)], eb, sm).wait()

                @plsc.parallel_loop(0, STEPS, unroll=8)
                def _(i, eb=eb):
                    pk16 = eb[pl.ds(i * 16, 16)]
                    col16 = pk16 & jnp.int32(0xFFFF)
                    rs = lax.shift_right_logical(pk16, jnp.int32(16))
                    for b in range(2):
                        gw = plsc.load_gather(
                            src.at[pl.ds(b * N, N)], [col16])
                        v0, v1 = plsc.unpack(
                            plsc.bitcast(gw, jnp.bfloat16),
                            format=plsc.PackFormat.INTERLEAVED)
                        plsc.addupdate_scatter(
                            dst.at[pl.ds(2 * b * N, N)], [rs], v0)
                        plsc.addupdate_scatter(
                            dst.at[pl.ds((2 * b + 1) * N, N)], [rs], v1)

                @pl.when(ci + 2 < NCHUNK)
                def _(eb=eb, sm=sm, ci=ci):
                    noff = pl.multiple_of((ci + 2) * CHUNK, 128)
                    pltpu.async_copy(
                        pk_ref.at[c, pl.ds(noff, CHUNK)], eb, sm)

            return carry

        lax.fori_loop(0, NCHUNK // 2, chunk_pair, 0)

        for bp in range(2):
            for b in range(2):
                pltpu.make_async_copy(
                    add_ref.at[w * CPW + 2 * bp + b], uxbufs[b],
                    uxsems[b]).wait()

            @plsc.parallel_loop(0, N // 16, unroll=5)
            def _(i, bp=bp):
                d = dinv[pl.ds(i * 16, 16)]
                sc = d if final else d * d
                s0 = dst[pl.ds(2 * bp * N + i * 16, 16)]
                s1 = dst[pl.ds((2 * bp + 1) * N + i * 16, 16)]
                v0 = sc * s0 + uxbufs[0][pl.ds(i * 16, 16)]
                v1 = sc * s1 + uxbufs[1][pl.ds(i * 16, 16)]
                if final:
                    dst[pl.ds(2 * bp * N + i * 16, 16)] = v0
                    dst[pl.ds((2 * bp + 1) * N + i * 16, 16)] = v1
                else:
                    src[pl.ds(bp * N + i * 16, 16)] = plsc.bitcast(
                        plsc.pack(v0, v1,
                                  format=plsc.PackFormat.INTERLEAVED),
                        jnp.int32)
                    dst[pl.ds(2 * bp * N + i * 16, 16)] = (
                        jnp.zeros((16,), jnp.float32))
                    dst[pl.ds((2 * bp + 1) * N + i * 16, 16)] = (
                        jnp.zeros((16,), jnp.float32))

            if bp == 0:
                for b in range(2):
                    pltpu.async_copy(add_ref.at[w * CPW + 2 + b],
                                     uxbufs[b], uxsems[b])

    def one_round(rr, carry):
        half_round(False)
        return carry

    lax.fori_loop(0, POWER1 - 1, one_round, 0)
    half_round(True)

    pltpu.sync_copy(buf_b, out_ref.at[w])


def _propagate(edge_index, xxr):
    mesh = plsc.VectorSubcoreMesh(core_axis_name="c", subcore_axis_name="s")
    f = pl.kernel(
        _prop_body,
        out_type=(
            jax.ShapeDtypeStruct((NW, CPW * N), jnp.float32),
            jax.ShapeDtypeStruct((2, E), jnp.int32),
            jax.ShapeDtypeStruct((D_HID, NPAD), jnp.float32),
        ),
        mesh=mesh,
        scratch_types=[
            pltpu.VMEM((2 * N,), jnp.int32),
            pltpu.VMEM((CPW * N,), jnp.float32),
            pltpu.VMEM((N,), jnp.float32),
            pltpu.VMEM((CHUNK,), jnp.int32),
            pltpu.VMEM((CHUNK,), jnp.int32),
            pltpu.VMEM((PKC,), jnp.int32),
            pltpu.VMEM((NPAD,), jnp.float32),
            pltpu.VMEM((NPAD,), jnp.float32),
            pltpu.SemaphoreType.DMA,
            pltpu.SemaphoreType.DMA,
            pltpu.SemaphoreType.DMA,
            pltpu.SemaphoreType.DMA,
        ],
        compiler_params=pltpu.CompilerParams(needs_layout_passes=False),
    )
    hxr, _, _ = f(edge_index, xxr)
    return hxr


def _head_body(xx_ref, hx_ref, pol_ref, wp_ref, bp_ref, o_ref):
    p0 = pol_ref[0]
    p1 = pol_ref[1]
    m = jnp.maximum(p0, p1)
    e0 = jnp.max(jnp.exp(jnp.full((8, 128), p0 - m, jnp.float32)))
    e1 = jnp.max(jnp.exp(jnp.full((8, 128), p1 - m, jnp.float32)))
    pp0 = e0 / (e0 + e1)
    pp1 = e1 / (e0 + e1)
    h = jnp.maximum(pp0 * xx_ref[...] + pp1 * hx_ref[...], 0.0)
    o_ref[...] = (
        jnp.dot(h, wp_ref[...], preferred_element_type=jnp.float32)
        + bp_ref[...]
    )


def _head(xX, hX, policy, W_pred, b_pred):
    return pl.pallas_call(
        _head_body,
        grid=(10,),
        in_specs=[
            pl.BlockSpec((N // 10, D_HID), lambda i: (i, 0)),
            pl.BlockSpec((N // 10, D_HID), lambda i: (i, 0)),
            pl.BlockSpec(memory_space=pltpu.SMEM),
            pl.BlockSpec((D_HID, D_OUT), lambda i: (0, 0)),
            pl.BlockSpec((1, D_OUT), lambda i: (0, 0)),
        ],
        out_specs=pl.BlockSpec((N // 10, D_OUT), lambda i: (i, 0)),
        out_shape=jax.ShapeDtypeStruct((N, D_OUT), jnp.float32),
    )(xX, hX, policy, W_pred, b_pred.reshape(1, D_OUT))


def kernel(x, edge_index, W_linX, b_linX, policy, W_pred, b_pred):
    xX = _lin_in(x, W_linX, b_linX)
    xxr = jnp.zeros((D_HID, NPAD), jnp.float32).at[:, :N].set(xX.T)
    hxr = _propagate(edge_index, xxr)
    hX = hxr.reshape(D_HID, N).T
    return _head(xX, hX, policy, W_pred, b_pred)


# speedup vs baseline: 1.0009x; 1.0009x over previous
import jax
import jax.numpy as jnp
from jax import lax
from jax.experimental import pallas as pl
from jax.experimental.pallas import tpu as pltpu
from jax.experimental.pallas import tpu_sc as plsc

N = 10000
E = 320000
D_IN = 128
D_HID = 128
D_OUT = 40
POWER1 = 8

NW = 32
NS = 16
CPW = D_HID // NW
CHUNK = 16000
NCHUNK = E // CHUNK
STEPS = CHUNK // 16
PKC = 2560
NPKC = E // PKC
NPAD = 10112


def _mm_in_body(x_ref, w_ref, b_ref, o_ref):
    o_ref[...] = (
        jnp.dot(x_ref[...], w_ref[...], preferred_element_type=jnp.float32)
        + b_ref[...]
    )


def _lin_in(x, W, b):
    return pl.pallas_call(
        _mm_in_body,
        grid=(10,),
        in_specs=[
            pl.BlockSpec((N // 10, D_IN), lambda i: (i, 0)),
            pl.BlockSpec((D_IN, D_HID), lambda i: (0, 0)),
            pl.BlockSpec((1, D_HID), lambda i: (0, 0)),
        ],
        out_specs=pl.BlockSpec((N // 10, D_HID), lambda i: (i, 0)),
        out_shape=jax.ShapeDtypeStruct((N, D_HID), jnp.float32),
    )(x, W, b.reshape(1, D_HID))


def _prop_body(edge_ref, xxr_ref, out_ref, pk_ref, uxr_ref,
               buf_a, buf_b, dinv, ebuf0, ebuf1, pout, uxbuf, uxbuf2,
               sem0, sem1, sem2, sem3):
    c = lax.axis_index("c")
    s = lax.axis_index("s")
    w = s * 2 + c
    ebufs = [ebuf0, ebuf1]
    sems = [sem0, sem1]
    uxbufs = [uxbuf, uxbuf2]
    uxsems = [sem2, sem3]

    @plsc.parallel_loop(0, N // 16, unroll=8)
    def _(i):
        dinv[pl.ds(i * 16, 16)] = jnp.zeros((16,), jnp.float32)

    ones = jnp.ones((16,), jnp.float32)

    pltpu.async_copy(edge_ref.at[1, pl.ds(0, CHUNK)], ebuf0, sem0)
    pltpu.async_copy(edge_ref.at[1, pl.ds(CHUNK, CHUNK)], ebuf1, sem1)

    def hist_pair(cj, carry):
        for b in range(2):
            ci = cj * 2 + b
            eb = ebufs[b]
            sm = sems[b]
            pltpu.make_async_copy(
                edge_ref.at[1, pl.ds(0, CHUNK)], eb, sm).wait()

            @plsc.parallel_loop(0, STEPS, unroll=16)
            def _(i, eb=eb):
                plsc.addupdate_scatter(dinv, [eb[pl.ds(i * 16, 16)]], ones)

            @pl.when(ci + 2 < NCHUNK)
            def _(eb=eb, sm=sm, ci=ci):
                noff = pl.multiple_of((ci + 2) * CHUNK, 128)
                pltpu.async_copy(
                    edge_ref.at[1, pl.ds(noff, CHUNK)], eb, sm)

        return carry

    lax.fori_loop(0, NCHUNK // 2, hist_pair, 0)

    pltpu.async_copy(edge_ref.at[0, pl.ds(0, CHUNK)], ebuf0, sem0)
    pltpu.async_copy(edge_ref.at[0, pl.ds(CHUNK, CHUNK)], ebuf1, sem1)

    def rmin_pair(cj, rminv):
        for b in range(2):
            ci = cj * 2 + b
            eb = ebufs[b]
            sm = sems[b]
            pltpu.make_async_copy(
                edge_ref.at[0, pl.ds(0, CHUNK)], eb, sm).wait()

            @plsc.parallel_loop(0, STEPS, step=8, carry=rminv)
            def rminv(i, rv, eb=eb):
                vs = [eb[pl.ds((i + k) * 16, 16)] for k in range(8)]
                for st in (4, 2, 1):
                    vs = [jnp.minimum(vs[k], vs[k + st])
                          for k in range(st)]
                return jnp.minimum(rv, vs[0])

            @pl.when(ci + 2 < NCHUNK)
            def _(eb=eb, sm=sm, ci=ci):
                noff = pl.multiple_of((ci + 2) * CHUNK, 128)
                pltpu.async_copy(
                    edge_ref.at[0, pl.ds(noff, CHUNK)], eb, sm)

        return rminv

    rminv = lax.fori_loop(
        0, NCHUNK // 2, rmin_pair,
        jnp.full((16,), jnp.iinfo(jnp.int32).max, jnp.int32),
    )
    rmin = jnp.min(rminv)

    @plsc.parallel_loop(0, N // 16, unroll=5)
    def _(i):
        d = dinv[pl.ds(i * 16, 16)]
        bits = plsc.bitcast(d, jnp.int32)
        y = plsc.bitcast(jnp.int32(0x5F3759DF) - (bits >> 1), jnp.float32)
        nh = d * jnp.float32(-0.5)
        for _ in range(3):
            y = y * (jnp.float32(1.5) + nh * y * y)
        dinv[pl.ds(i * 16, 16)] = jnp.where(d > 0.0, y, jnp.float32(0.0))

    cps = -(-NPKC // NS)
    npk = jnp.minimum(cps, NPKC - s * cps)

    def pack_chunk(k, carry):
        off = pl.multiple_of((s * cps + k) * PKC, 128)
        pltpu.sync_copy(edge_ref.at[0, pl.ds(off, PKC)],
                        ebuf0.at[pl.ds(0, PKC)])
        pltpu.sync_copy(edge_ref.at[1, pl.ds(off, PKC)],
                        ebuf1.at[pl.ds(0, PKC)])

        @plsc.parallel_loop(0, PKC // 16, unroll=4)
        def _(i):
            row16 = ebuf0[pl.ds(i * 16, 16)] - rmin
            col16 = ebuf1[pl.ds(i * 16, 16)]
            pout[pl.ds(i * 16, 16)] = (row16 << 16) | col16

        pltpu.sync_copy(pout, pk_ref.at[c, pl.ds(off, PKC)])
        return carry

    lax.fori_loop(0, npk, pack_chunk, 0)
    plsc.subcore_barrier()

    for b in range(2):
        pltpu.sync_copy(xxr_ref.at[w * CPW + 2 * b], uxbuf)
        pltpu.sync_copy(xxr_ref.at[w * CPW + 2 * b + 1], uxbuf2)

        @plsc.parallel_loop(0, N // 16, unroll=4)
        def _(i, b=b):
            d = dinv[pl.ds(i * 16, 16)]
            v0 = uxbuf[pl.ds(i * 16, 16)] * d
            v1 = uxbuf2[pl.ds(i * 16, 16)] * d
            uxbuf[pl.ds(i * 16, 16)] = v0
            uxbuf2[pl.ds(i * 16, 16)] = v1
            buf_a[pl.ds(b * N + i * 16, 16)] = plsc.bitcast(
                plsc.pack(v0, v1, format=plsc.PackFormat.INTERLEAVED),
                jnp.int32)

        pltpu.sync_copy(uxbuf, uxr_ref.at[w * CPW + 2 * b])
        pltpu.sync_copy(uxbuf2, uxr_ref.at[w * CPW + 2 * b + 1])

    @plsc.parallel_loop(0, CPW * N // 16, unroll=8)
    def _(i):
        buf_b[pl.ds(i * 16, 16)] = jnp.zeros((16,), jnp.float32)

    def half_round(final):
        src = buf_a
        dst = buf_b
        add_ref = xxr_ref if final else uxr_ref
        for j in range(2):
            pltpu.async_copy(add_ref.at[w * CPW + j], uxbufs[j], uxsems[j])
        pltpu.async_copy(pk_ref.at[c, pl.ds(0, CHUNK)], ebuf0, sem0)
        pltpu.async_copy(pk_ref.at[c, pl.ds(CHUNK, CHUNK)], ebuf1, sem1)

        def chunk_pair(cj, carry):
            for b in range(2):
                ci = cj * 2 + b
                eb = ebufs[b]
                sm = sems[b]
                pltpu.make_async_copy(
                    pk_ref.at[c, pl.ds(0, CHUNK)], eb, sm).wait()

                @plsc.parallel_loop(0, STEPS, unroll=8)
                def _(i, eb=eb):
                    pk16 = eb[pl.ds(i * 16, 16)]
                    col16 = pk16 & jnp.int32(0xFFFF)
                    rs = lax.shift_right_logical(pk16, jnp.int32(16))
                    for b in range(2):
                        gw = plsc.load_gather(
                            src.at[pl.ds(b * N, N)], [col16])
                        v0, v1 = plsc.unpack(
                            plsc.bitcast(gw, jnp.bfloat16),
                            format=plsc.PackFormat.INTERLEAVED)
                        plsc.addupdate_scatter(
                            dst.at[pl.ds(2 * b * N, N)], [rs], v0)
                        plsc.addupdate_scatter(
                            dst.at[pl.ds((2 * b + 1) * N, N)], [rs], v1)

                @pl.when(ci + 2 < NCHUNK)
                def _(eb=eb, sm=sm, ci=ci):
                    noff = pl.multiple_of((ci + 2) * CHUNK, 128)
                    pltpu.async_copy(
                        pk_ref.at[c, pl.ds(noff, CHUNK)], eb, sm)

            return carry

        lax.fori_loop(0, NCHUNK // 2, chunk_pair, 0)

        for bp in range(2):
            for b in range(2):
                pltpu.make_async_copy(
                    add_ref.at[w * CPW + 2 * bp + b], uxbufs[b],
                    uxsems[b]).wait()

            @plsc.parallel_loop(0, N // 16, unroll=5)
            def _(i, bp=bp):
                d = dinv[pl.ds(i * 16, 16)]
                sc = d if final else d * d
                s0 = dst[pl.ds(2 * bp * N + i * 16, 16)]
                s1 = dst[pl.ds((2 * bp + 1) * N + i * 16, 16)]
                v0 = sc * s0 + uxbufs[0][pl.ds(i * 16, 16)]
                v1 = sc * s1 + uxbufs[1][pl.ds(i * 16, 16)]
                if final:
                    dst[pl.ds(2 * bp * N + i * 16, 16)] = v0
                    dst[pl.ds((2 * bp + 1) * N + i * 16, 16)] = v1
                else:
                    src[pl.ds(bp * N + i * 16, 16)] = plsc.bitcast(
                        plsc.pack(v0, v1,
                                  format=plsc.PackFormat.INTERLEAVED),
                        jnp.int32)
                    dst[pl.ds(2 * bp * N + i * 16, 16)] = (
                        jnp.zeros((16,), jnp.float32))
                    dst[pl.ds((2 * bp + 1) * N + i * 16, 16)] = (
                        jnp.zeros((16,), jnp.float32))

            if bp == 0:
                for b in range(2):
                    pltpu.async_copy(add_ref.at[w * CPW + 2 + b],
                                     uxbufs[b], uxsems[b])

    def one_round(rr, carry):
        half_round(False)
        return carry

    lax.fori_loop(0, POWER1 - 1, one_round, 0)
    half_round(True)

    pltpu.sync_copy(buf_b, out_ref.at[w])


def _propagate(edge_index, xxr):
    mesh = plsc.VectorSubcoreMesh(core_axis_name="c", subcore_axis_name="s")
    f = pl.kernel(
        _prop_body,
        out_type=(
            jax.ShapeDtypeStruct((NW, CPW * N), jnp.float32),
            jax.ShapeDtypeStruct((2, E), jnp.int32),
            jax.ShapeDtypeStruct((D_HID, NPAD), jnp.float32),
        ),
        mesh=mesh,
        scratch_types=[
            pltpu.VMEM((2 * N,), jnp.int32),
            pltpu.VMEM((CPW * N,), jnp.float32),
            pltpu.VMEM((N,), jnp.float32),
            pltpu.VMEM((CHUNK,), jnp.int32),
            pltpu.VMEM((CHUNK,), jnp.int32),
            pltpu.VMEM((PKC,), jnp.int32),
            pltpu.VMEM((NPAD,), jnp.float32),
            pltpu.VMEM((NPAD,), jnp.float32),
            pltpu.SemaphoreType.DMA,
            pltpu.SemaphoreType.DMA,
            pltpu.SemaphoreType.DMA,
            pltpu.SemaphoreType.DMA,
        ],
        compiler_params=pltpu.CompilerParams(needs_layout_passes=False),
    )
    hxr, _, _ = f(edge_index, xxr)
    return hxr


def _head_body(xx_ref, hx_ref, pol_ref, wp_ref, bp_ref, o_ref):
    p0 = pol_ref[0]
    p1 = pol_ref[1]
    m = jnp.maximum(p0, p1)
    e0 = jnp.max(jnp.exp(jnp.full((8, 128), p0 - m, jnp.float32)))
    e1 = jnp.max(jnp.exp(jnp.full((8, 128), p1 - m, jnp.float32)))
    pp0 = e0 / (e0 + e1)
    pp1 = e1 / (e0 + e1)
    h = jnp.maximum(pp0 * xx_ref[...] + pp1 * hx_ref[...], 0.0)
    o_ref[...] = (
        jnp.dot(h, wp_ref[...], preferred_element_type=jnp.float32)
        + bp_ref[...]
    )


def _head(xX, hX, policy, W_pred, b_pred):
    return pl.pallas_call(
        _head_body,
        grid=(10,),
        in_specs=[
            pl.BlockSpec((N // 10, D_HID), lambda i: (i, 0)),
            pl.BlockSpec((N // 10, D_HID), lambda i: (i, 0)),
            pl.BlockSpec(memory_space=pltpu.SMEM),
            pl.BlockSpec((D_HID, D_OUT), lambda i: (0, 0)),
            pl.BlockSpec((1, D_OUT), lambda i: (0, 0)),
        ],
        out_specs=pl.BlockSpec((N // 10, D_OUT), lambda i: (i, 0)),
        out_shape=jax.ShapeDtypeStruct((N, D_OUT), jnp.float32),
    )(xX, hX, policy, W_pred, b_pred.reshape(1, D_OUT))


def kernel(x, edge_index, W_linX, b_linX, policy, W_pred, b_pred):
    xX = _lin_in(x, W_linX, b_linX)
    xxr = jnp.zeros((D_HID, NPAD), jnp.float32).at[:, :N].set(xX.T)
    hxr = _propagate(edge_index, xxr)
    hX = hxr.reshape(D_HID, N).T
    return _head(xX, hX, policy, W_pred, b_pred)
